# baseline (speedup 1.0000x reference)
import functools

import jax
import jax.numpy as jnp
from jax import lax
from jax.experimental import pallas as pl
from jax.experimental.pallas import tpu as pltpu
from jax.experimental.pallas import tpu_sc as plsc

E = 160000
T = 160000
H = 128
NB = 8

NC = 2
NS = 16
NW = NC * NS

GB = 128
GNB = T // GB

_sc_mesh = dict(
    mesh=plsc.VectorSubcoreMesh(core_axis_name="c", subcore_axis_name="s"),
    compiler_params=pltpu.CompilerParams(needs_layout_passes=False),
)


@functools.partial(
    pl.kernel,
    out_type=jax.ShapeDtypeStruct((T, H), jnp.float32),
    scratch_types=[
        pltpu.VMEM((2, GB), jnp.int32),
        pltpu.VMEM((2, GB, H), jnp.float32),
        pltpu.SemaphoreType.DMA,
        pltpu.SemaphoreType.DMA,
    ],
    **_sc_mesh,
)
def _sc_gather(p_hbm, kj_hbm, pk_out, idxb, rows, gsem, wsem):
    wid = lax.axis_index("s") * NC + lax.axis_index("c")
    nb = jnp.where(wid < GNB % NW, GNB // NW + 1, GNB // NW)

    def outer(k2, _):
        hs = []
        for b in range(2):
            k = k2 * 2 + b
            o = pl.multiple_of((wid + k * NW) * GB, 8)
            pltpu.sync_copy(kj_hbm.at[pl.ds(o, GB)], idxb.at[b])
            hs.append(pltpu.async_copy(p_hbm.at[idxb.at[b]], rows.at[b], gsem))
        ws = []
        for b in range(2):
            k = k2 * 2 + b
            o = pl.multiple_of((wid + k * NW) * GB, 8)
            hs[b].wait()
            ws.append(pltpu.async_copy(rows.at[b], pk_out.at[pl.ds(o, GB)], wsem))
        for w in ws:
            w.wait()
        return 0

    lax.fori_loop(0, nb // 2, outer, 0)

    def tail(k, _):
        o = pl.multiple_of((wid + k * NW) * GB, 8)
        pltpu.sync_copy(kj_hbm.at[pl.ds(o, GB)], idxb.at[0])
        pltpu.async_copy(p_hbm.at[idxb.at[0]], rows.at[0], gsem).wait()
        pltpu.sync_copy(rows.at[0], pk_out.at[pl.ds(o, GB)])
        return 0

    lax.fori_loop(nb // 2 * 2, nb, tail, 0)


CHUNK = 6400
NCHUNK = E // CHUNK
NCITER = (NCHUNK + 1) // NC
TPW = T // NS
SLICE = CHUNK // NS
SB = 128
DUMMY = CHUNK


@functools.partial(
    pl.kernel,
    out_type=jax.ShapeDtypeStruct((E, H), jnp.float32),
    scratch_types=[
        pltpu.VMEM((TPW,), jnp.int32),
        pltpu.VMEM((TPW + SB,), jnp.int32),
        pltpu.VMEM((TPW + SB,), jnp.int32),
        pltpu.VMEM((2, SB), jnp.int32),
        pltpu.VMEM((2, SB), jnp.int32),
        pltpu.VMEM((2, SB, H), jnp.float32),
        pltpu.VMEM((64, H), jnp.float32),
        pltpu.VMEM_SHARED((CHUNK + 8, H), jnp.float32),
        pltpu.SemaphoreType.DMA,
        pltpu.SemaphoreType.DMA,
    ],
    **_sc_mesh,
)
def _sc_scatter(tmsg_hbm, ji2_hbm, out_hbm, jibuf, tbuf, dbuf, tidx, didx,
                rows, zbuf, spc, sem0, sem1):
    cid = lax.axis_index("c")
    sid = lax.axis_index("s")
    t0 = sid * TPW
    pltpu.sync_copy(ji2_hbm.at[sid], jibuf)

    zv = jnp.zeros((16,), jnp.float32)

    def zrow(r, _):
        for c in range(8):
            zbuf[r, pl.ds(c * 16, 16)] = zv
        return 0

    lax.fori_loop(0, 64, zrow, 0)
    lanes = lax.iota(jnp.int32, 16)

    def chunk_body(ci, _):
        chunk = ci * NC + cid
        start = chunk * CHUNK
        valid = chunk < NCHUNK

        @pl.when(valid)
        def _zero():
            for z in range(6):
                pltpu.sync_copy(zbuf, spc.at[pl.ds(sid * SLICE + z * 64, 64)])
            pltpu.sync_copy(zbuf.at[pl.ds(0, SLICE - 6 * 64)],
                            spc.at[pl.ds(sid * SLICE + 6 * 64, SLICE - 6 * 64)])

        plsc.subcore_barrier()

        def fbody(i, cnt_vec):
            for u in range(5):
                ii = i * 5 + u
                v = jibuf[pl.ds(ii * 16, 16)]
                msk = (v >= start) & (v < start + CHUNK)
                inc = plsc.cumsum(msk.astype(jnp.int32))
                pos = (cnt_vec - 1) + inc
                plsc.store_scatter(tbuf, [pos], (t0 + ii * 16) + lanes,
                                   mask=msk)
                plsc.store_scatter(dbuf, [pos], v - start, mask=msk)
                cnt_vec = cnt_vec + plsc.all_reduce_population_count(msk)
            return cnt_vec

        cnt_vec = lax.fori_loop(0, TPW // 16 // 5, fbody,
                                jnp.zeros((16,), jnp.int32))
        cnt = jnp.max(cnt_vec)
        pad_base = lax.rem(t0 + TPW, jnp.int32(T))
        ones = lanes >= 0
        for k in range(SB // 16):
            pos = cnt_vec + (k * 16) + lanes
            plsc.store_scatter(tbuf, [pos], pad_base + (k * 16) + lanes,
                               mask=ones)
            plsc.store_scatter(dbuf, [pos], jnp.full((16,), DUMMY, jnp.int32),
                               mask=ones)

        nbp = jnp.maximum((cnt + SB - 1) // SB, 1)
        sems = (sem0, sem1)

        def prep(b, j):
            o = pl.multiple_of(j * SB, 8)
            for k in range(SB // 16):
                tidx[b, pl.ds(k * 16, 16)] = tbuf[pl.ds(o + k * 16, 16)]
                didx[b, pl.ds(k * 16, 16)] = dbuf[pl.ds(o + k * 16, 16)]

        def fire(b, j):
            prep(b, j)
            pltpu.async_copy(tmsg_hbm.at[tidx.at[b]], rows.at[b], sems[b])

        def drain_accum(b):
            pltpu.make_async_copy(tmsg_hbm.at[pl.ds(0, SB)], rows.at[b],
                                  sems[b]).wait()
            pltpu.sync_copy(rows.at[b], spc.at[didx.at[b]], add=True)

        fire(0, jnp.int32(0))

        def bpair(q, _):
            j1 = q * 2 + 1

            @pl.when(j1 < nbp)
            def _fire1():
                fire(1, j1)

            drain_accum(0)

            @pl.when(j1 + 1 < nbp)
            def _fire0():
                fire(0, j1 + 1)

            @pl.when(j1 < nbp)
            def _drain1():
                drain_accum(1)

            return 0

        lax.fori_loop(0, (nbp + 1) // 2, bpair, 0)
        plsc.subcore_barrier()

        @pl.when(valid)
        def _writeout():
            pltpu.sync_copy(
                spc.at[pl.ds(sid * SLICE, SLICE)],
                out_hbm.at[pl.ds(pl.multiple_of(start + sid * SLICE, 8),
                                 SLICE)])

        return 0

    lax.fori_loop(0, NCITER, chunk_body, 0)


TM = 1000


def _silu(x):
    return x * (1.0 / (1.0 + jnp.exp(-x)))


def _tc_pre_body(m, rbf, w_kj, b_kj, w_rbf, p):
    x = _silu(jnp.dot(m[:].astype(jnp.bfloat16), w_kj[:],
                      preferred_element_type=jnp.float32) + b_kj[:])
    gate = jnp.dot(rbf[:], w_rbf[:], preferred_element_type=jnp.float32)
    p[:] = x * gate


def _tc_pre(m, rbf, w_kj, b_kj, w_rbf):
    full = lambda *shape: pl.BlockSpec(shape, lambda i: tuple(0 for _ in shape))
    return pl.pallas_call(
        _tc_pre_body,
        grid=(E // TM,),
        in_specs=[
            pl.BlockSpec((TM, H), lambda i: (i, 0)),
            pl.BlockSpec((TM, 6), lambda i: (i, 0)),
            full(H, H), full(1, H), full(6, H),
        ],
        out_specs=pl.BlockSpec((TM, H), lambda i: (i, 0)),
        out_shape=jax.ShapeDtypeStruct((E, H), jnp.float32),
    )(m, rbf, w_kj, b_kj, w_rbf)


def _tc_triplet_body(pk, sbf, w_sbf, wb, tmsg):
    sp = jnp.dot(sbf[:], w_sbf[:], preferred_element_type=jnp.float32)
    xk = pk[:].astype(jnp.bfloat16)
    acc = jnp.zeros((TM, H), jnp.float32)
    for b in range(NB):
        acc = acc + jnp.dot(xk, wb[b], preferred_element_type=jnp.float32) \
            * sp[:, b:b + 1]
    tmsg[:] = acc


def _tc_triplet(pk, sbf, w_sbf, wb):
    full = lambda *shape: pl.BlockSpec(shape, lambda i: tuple(0 for _ in shape))
    return pl.pallas_call(
        _tc_triplet_body,
        grid=(T // TM,),
        in_specs=[
            pl.BlockSpec((TM, H), lambda i: (i, 0)),
            pl.BlockSpec((TM, 42), lambda i: (i, 0)),
            full(42, NB), full(NB, H, H),
        ],
        out_specs=pl.BlockSpec((TM, H), lambda i: (i, 0)),
        out_shape=jax.ShapeDtypeStruct((T, H), jnp.float32),
    )(pk, sbf, w_sbf, wb)


def _bdot(a, w):
    return jnp.dot(a.astype(jnp.bfloat16), w,
                   preferred_element_type=jnp.float32)


def _tc_post_body(m, aggr, w_ji, b_ji, r1w1, r1b1, r1w2, r1b2,
                  r2w1, r2b1, r2w2, r2b2, w_out, b_out, out):
    mv = m[:]
    x = _silu(_bdot(mv, w_ji[:]) + b_ji[:]) + aggr[:]
    h = _silu(x)
    h = _silu(_bdot(h, r1w1[:]) + r1b1[:])
    h = _bdot(h, r1w2[:]) + r1b2[:]
    x = x + h
    h = _silu(x)
    h = _silu(_bdot(h, r2w1[:]) + r2b1[:])
    h = _bdot(h, r2w2[:]) + r2b2[:]
    x = x + h
    out[:] = mv + _silu(_bdot(x, w_out[:]) + b_out[:])


def _tc_post(m, aggr, w_ji, b_ji, r1w1, r1b1, r1w2, r1b2,
             r2w1, r2b1, r2w2, r2b2, w_out, b_out):
    mat = pl.BlockSpec((H, H), lambda i: (0, 0))
    vec = pl.BlockSpec((1, H), lambda i: (0, 0))
    return pl.pallas_call(
        _tc_post_body,
        grid=(E // TM,),
        in_specs=[
            pl.BlockSpec((TM, H), lambda i: (i, 0)),
            pl.BlockSpec((TM, H), lambda i: (i, 0)),
            mat, vec, mat, vec, mat, vec, mat, vec, mat, vec, mat, vec,
        ],
        out_specs=pl.BlockSpec((TM, H), lambda i: (i, 0)),
        out_shape=jax.ShapeDtypeStruct((E, H), jnp.float32),
    )(m, aggr, w_ji, b_ji, r1w1, r1b1, r1w2, r1b2,
      r2w1, r2b1, r2w2, r2b2, w_out, b_out)


def kernel(m, rbf, sbf, edge_index, triplet_index, W_rbf, W_sbf, W_kj, b_kj,
           W_ji, b_ji, W_bil, r1_w1, r1_b1, r1_w2, r1_b2, r2_w1, r2_b1,
           r2_w2, r2_b2, W_out, b_out):
    bf16 = jnp.bfloat16
    kj = triplet_index[0].astype(jnp.int32)
    ji = triplet_index[1].astype(jnp.int32)
    ji2 = ji.reshape(NS, TPW)
    wb = jnp.transpose(W_bil, (1, 0, 2)).astype(bf16)

    p = _tc_pre(m, rbf, W_kj.astype(bf16), b_kj.reshape(1, H), W_rbf)
    pk = _sc_gather(p, kj)
    tmsg = _tc_triplet(pk, sbf, W_sbf, wb)
    aggr = _sc_scatter(tmsg, ji2)
    return _tc_post(m, aggr, W_ji.astype(bf16), b_ji.reshape(1, H),
                    r1_w1.astype(bf16), r1_b1.reshape(1, H),
                    r1_w2.astype(bf16), r1_b2.reshape(1, H),
                    r2_w1.astype(bf16), r2_b1.reshape(1, H),
                    r2_w2.astype(bf16), r2_b2.reshape(1, H),
                    W_out.astype(bf16), b_out.reshape(1, H))


# speedup vs baseline: 1.2289x; 1.2289x over previous
import functools

import jax
import jax.numpy as jnp
from jax import lax
from jax.experimental import pallas as pl
from jax.experimental.pallas import tpu as pltpu
from jax.experimental.pallas import tpu_sc as plsc

E = 160000
T = 160000
H = 128
NB = 8

NC = 2
NS = 16
NW = NC * NS

GB = 128
GNB = T // GB

_sc_mesh = dict(
    mesh=plsc.VectorSubcoreMesh(core_axis_name="c", subcore_axis_name="s"),
    compiler_params=pltpu.CompilerParams(needs_layout_passes=False),
)


@functools.partial(
    pl.kernel,
    out_type=jax.ShapeDtypeStruct((T, H), jnp.float32),
    scratch_types=[
        pltpu.VMEM((2, GB), jnp.int32),
        pltpu.VMEM((2, GB, H), jnp.float32),
        pltpu.SemaphoreType.DMA,
        pltpu.SemaphoreType.DMA,
    ],
    **_sc_mesh,
)
def _sc_gather(p_hbm, kj_hbm, pk_out, idxb, rows, gsem, wsem):
    wid = lax.axis_index("s") * NC + lax.axis_index("c")
    nb = jnp.where(wid < GNB % NW, GNB // NW + 1, GNB // NW)

    def outer(k2, _):
        hs = []
        for b in range(2):
            k = k2 * 2 + b
            o = pl.multiple_of((wid + k * NW) * GB, 8)
            pltpu.sync_copy(kj_hbm.at[pl.ds(o, GB)], idxb.at[b])
            hs.append(pltpu.async_copy(p_hbm.at[idxb.at[b]], rows.at[b], gsem))
        ws = []
        for b in range(2):
            k = k2 * 2 + b
            o = pl.multiple_of((wid + k * NW) * GB, 8)
            hs[b].wait()
            ws.append(pltpu.async_copy(rows.at[b], pk_out.at[pl.ds(o, GB)], wsem))
        for w in ws:
            w.wait()
        return 0

    lax.fori_loop(0, nb // 2, outer, 0)

    def tail(k, _):
        o = pl.multiple_of((wid + k * NW) * GB, 8)
        pltpu.sync_copy(kj_hbm.at[pl.ds(o, GB)], idxb.at[0])
        pltpu.async_copy(p_hbm.at[idxb.at[0]], rows.at[0], gsem).wait()
        pltpu.sync_copy(rows.at[0], pk_out.at[pl.ds(o, GB)])
        return 0

    lax.fori_loop(nb // 2 * 2, nb, tail, 0)


CHUNK = 6400
NCHUNK = E // CHUNK
NCITER = (NCHUNK + 1) // NC
TPW = T // NS
SLICE = CHUNK // NS
SB = 128
DUMMY = CHUNK


@functools.partial(
    pl.kernel,
    out_type=jax.ShapeDtypeStruct((E, H), jnp.float32),
    scratch_types=[
        pltpu.VMEM((TPW,), jnp.int32),
        pltpu.VMEM((TPW + SB,), jnp.int32),
        pltpu.VMEM((2, SB), jnp.int32),
        pltpu.VMEM((2, SB), jnp.int32),
        pltpu.VMEM((2, SB, H), jnp.float32),
        pltpu.VMEM((64, H), jnp.float32),
        pltpu.VMEM_SHARED((CHUNK + 8, H), jnp.float32),
        pltpu.SemaphoreType.DMA,
        pltpu.SemaphoreType.DMA,
    ],
    **_sc_mesh,
)
def _sc_scatter(tmsg_hbm, ji2_hbm, out_hbm, jibuf, pbuf, tidx, didx,
                rows, zbuf, spc, sem0, sem1):
    cid = lax.axis_index("c")
    sid = lax.axis_index("s")
    t0 = sid * TPW
    pltpu.sync_copy(ji2_hbm.at[sid], jibuf)

    zv = jnp.zeros((16,), jnp.float32)

    def zrow(r, _):
        for c in range(8):
            zbuf[r, pl.ds(c * 16, 16)] = zv
        return 0

    lax.fori_loop(0, 64, zrow, 0)
    lanes = lax.iota(jnp.int32, 16)

    def chunk_body(ci, _):
        chunk = ci * NC + cid
        start = chunk * CHUNK
        valid = chunk < NCHUNK

        @pl.when(valid)
        def _zero():
            for z in range(6):
                pltpu.sync_copy(zbuf, spc.at[pl.ds(sid * SLICE + z * 64, 64)])
            pltpu.sync_copy(zbuf.at[pl.ds(0, SLICE - 6 * 64)],
                            spc.at[pl.ds(sid * SLICE + 6 * 64, SLICE - 6 * 64)])

        plsc.subcore_barrier()

        def fbody(i, cnt_vec):
            for u in range(5):
                ii = i * 5 + u
                v = jibuf[pl.ds(ii * 16, 16)]
                msk = (v >= start) & (v < start + CHUNK)
                inc = plsc.cumsum(msk.astype(jnp.int32))
                pos = (cnt_vec - 1) + inc
                packed = ((t0 + ii * 16) + lanes) * 8192 + (v - start)
                plsc.store_scatter(pbuf, [pos], packed, mask=msk)
                cnt_vec = cnt_vec + plsc.all_reduce_population_count(msk)
            return cnt_vec

        cnt_vec = lax.fori_loop(0, TPW // 16 // 5, fbody,
                                jnp.zeros((16,), jnp.int32))
        cnt = jnp.max(cnt_vec)
        pad_base = lax.rem(t0 + TPW, jnp.int32(T))
        ones = lanes >= 0
        for k in range(SB // 16):
            pos = cnt_vec + (k * 16) + lanes
            plsc.store_scatter(
                pbuf, [pos],
                (pad_base + (k * 16) + lanes) * 8192 + DUMMY, mask=ones)

        nbp = jnp.maximum((cnt + SB - 1) // SB, 1)
        sems = (sem0, sem1)

        def prep(b, j):
            o = pl.multiple_of(j * SB, 8)
            for k in range(SB // 16):
                w = pbuf[pl.ds(o + k * 16, 16)]
                tidx[b, pl.ds(k * 16, 16)] = jnp.right_shift(w, 13)
                didx[b, pl.ds(k * 16, 16)] = jnp.bitwise_and(w, 8191)

        def fire(b, j):
            prep(b, j)
            pltpu.async_copy(tmsg_hbm.at[tidx.at[b]], rows.at[b], sems[b])

        def drain_accum(b):
            pltpu.make_async_copy(tmsg_hbm.at[pl.ds(0, SB)], rows.at[b],
                                  sems[b]).wait()
            pltpu.sync_copy(rows.at[b], spc.at[didx.at[b]], add=True)

        fire(0, jnp.int32(0))

        def bpair(q, _):
            j1 = q * 2 + 1

            @pl.when(j1 < nbp)
            def _fire1():
                fire(1, j1)

            drain_accum(0)

            @pl.when(j1 + 1 < nbp)
            def _fire0():
                fire(0, j1 + 1)

            @pl.when(j1 < nbp)
            def _drain1():
                drain_accum(1)

            return 0

        lax.fori_loop(0, (nbp + 1) // 2, bpair, 0)
        plsc.subcore_barrier()

        @pl.when(valid)
        def _writeout():
            pltpu.sync_copy(
                spc.at[pl.ds(sid * SLICE, SLICE)],
                out_hbm.at[pl.ds(pl.multiple_of(start + sid * SLICE, 8),
                                 SLICE)])

        return 0

    lax.fori_loop(0, NCITER, chunk_body, 0)


TM = 4000


def _silu(x):
    return x * (1.0 / (1.0 + jnp.exp(-x)))


def _tc_pre_body(m, rbf, w_kj, b_kj, w_rbf, p):
    x = _silu(jnp.dot(m[:].astype(jnp.bfloat16), w_kj[:],
                      preferred_element_type=jnp.float32) + b_kj[:])
    gate = jnp.dot(rbf[:], w_rbf[:], preferred_element_type=jnp.float32)
    p[:] = x * gate


def _tc_pre(m, rbf, w_kj, b_kj, w_rbf):
    full = lambda *shape: pl.BlockSpec(shape, lambda i: tuple(0 for _ in shape))
    return pl.pallas_call(
        _tc_pre_body,
        grid=(E // TM,),
        in_specs=[
            pl.BlockSpec((TM, H), lambda i: (i, 0)),
            pl.BlockSpec((TM, 6), lambda i: (i, 0)),
            full(H, H), full(1, H), full(6, H),
        ],
        out_specs=pl.BlockSpec((TM, H), lambda i: (i, 0)),
        out_shape=jax.ShapeDtypeStruct((E, H), jnp.float32),
    )(m, rbf, w_kj, b_kj, w_rbf)


def _tc_triplet_body(pk, sbf, w_sbf, wb, tmsg):
    sp = jnp.dot(sbf[:], w_sbf[:], preferred_element_type=jnp.float32)
    xk = pk[:].astype(jnp.bfloat16)
    acc = jnp.zeros((TM, H), jnp.float32)
    for b in range(NB):
        acc = acc + jnp.dot(xk, wb[b], preferred_element_type=jnp.float32) \
            * sp[:, b:b + 1]
    tmsg[:] = acc


def _tc_triplet(pk, sbf, w_sbf, wb):
    full = lambda *shape: pl.BlockSpec(shape, lambda i: tuple(0 for _ in shape))
    return pl.pallas_call(
        _tc_triplet_body,
        grid=(T // TM,),
        in_specs=[
            pl.BlockSpec((TM, H), lambda i: (i, 0)),
            pl.BlockSpec((TM, 42), lambda i: (i, 0)),
            full(42, NB), full(NB, H, H),
        ],
        out_specs=pl.BlockSpec((TM, H), lambda i: (i, 0)),
        out_shape=jax.ShapeDtypeStruct((T, H), jnp.float32),
    )(pk, sbf, w_sbf, wb)


def _bdot(a, w):
    return jnp.dot(a.astype(jnp.bfloat16), w,
                   preferred_element_type=jnp.float32)


def _tc_post_body(m, aggr, w_ji, b_ji, r1w1, r1b1, r1w2, r1b2,
                  r2w1, r2b1, r2w2, r2b2, w_out, b_out, out):
    mv = m[:]
    x = _silu(_bdot(mv, w_ji[:]) + b_ji[:]) + aggr[:]
    h = _silu(x)
    h = _silu(_bdot(h, r1w1[:]) + r1b1[:])
    h = _bdot(h, r1w2[:]) + r1b2[:]
    x = x + h
    h = _silu(x)
    h = _silu(_bdot(h, r2w1[:]) + r2b1[:])
    h = _bdot(h, r2w2[:]) + r2b2[:]
    x = x + h
    out[:] = mv + _silu(_bdot(x, w_out[:]) + b_out[:])


def _tc_post(m, aggr, w_ji, b_ji, r1w1, r1b1, r1w2, r1b2,
             r2w1, r2b1, r2w2, r2b2, w_out, b_out):
    mat = pl.BlockSpec((H, H), lambda i: (0, 0))
    vec = pl.BlockSpec((1, H), lambda i: (0, 0))
    return pl.pallas_call(
        _tc_post_body,
        grid=(E // TM,),
        in_specs=[
            pl.BlockSpec((TM, H), lambda i: (i, 0)),
            pl.BlockSpec((TM, H), lambda i: (i, 0)),
            mat, vec, mat, vec, mat, vec, mat, vec, mat, vec, mat, vec,
        ],
        out_specs=pl.BlockSpec((TM, H), lambda i: (i, 0)),
        out_shape=jax.ShapeDtypeStruct((E, H), jnp.float32),
    )(m, aggr, w_ji, b_ji, r1w1, r1b1, r1w2, r1b2,
      r2w1, r2b1, r2w2, r2b2, w_out, b_out)


def kernel(m, rbf, sbf, edge_index, triplet_index, W_rbf, W_sbf, W_kj, b_kj,
           W_ji, b_ji, W_bil, r1_w1, r1_b1, r1_w2, r1_b2, r2_w1, r2_b1,
           r2_w2, r2_b2, W_out, b_out):
    bf16 = jnp.bfloat16
    kj = triplet_index[0].astype(jnp.int32)
    ji = triplet_index[1].astype(jnp.int32)
    ji2 = ji.reshape(NS, TPW)
    wb = jnp.transpose(W_bil, (1, 0, 2)).astype(bf16)

    p = _tc_pre(m, rbf, W_kj.astype(bf16), b_kj.reshape(1, H), W_rbf)
    pk = _sc_gather(p, kj)
    tmsg = _tc_triplet(pk, sbf, W_sbf, wb)
    aggr = _sc_scatter(tmsg, ji2)
    return _tc_post(m, aggr, W_ji.astype(bf16), b_ji.reshape(1, H),
                    r1_w1.astype(bf16), r1_b1.reshape(1, H),
                    r1_w2.astype(bf16), r1_b2.reshape(1, H),
                    r2_w1.astype(bf16), r2_b1.reshape(1, H),
                    r2_w2.astype(bf16), r2_b2.reshape(1, H),
                    W_out.astype(bf16), b_out.reshape(1, H))


# speedup vs baseline: 1.3402x; 1.0906x over previous
import functools

import jax
import jax.numpy as jnp
from jax import lax
from jax.experimental import pallas as pl
from jax.experimental.pallas import tpu as pltpu
from jax.experimental.pallas import tpu_sc as plsc

E = 160000
T = 160000
H = 128
NB = 8

NC = 2
NS = 16
NW = NC * NS

GB = 128
GNB = T // GB

_sc_mesh = dict(
    mesh=plsc.VectorSubcoreMesh(core_axis_name="c", subcore_axis_name="s"),
    compiler_params=pltpu.CompilerParams(needs_layout_passes=False),
)


@functools.partial(
    pl.kernel,
    out_type=jax.ShapeDtypeStruct((T, H), jnp.float32),
    scratch_types=[
        pltpu.VMEM((2, GB), jnp.int32),
        pltpu.VMEM((2, GB, H), jnp.float32),
        pltpu.SemaphoreType.DMA,
        pltpu.SemaphoreType.DMA,
    ],
    **_sc_mesh,
)
def _sc_gather(p_hbm, kj_hbm, pk_out, idxb, rows, gsem, wsem):
    wid = lax.axis_index("s") * NC + lax.axis_index("c")
    nb = jnp.where(wid < GNB % NW, GNB // NW + 1, GNB // NW)

    def outer(k2, _):
        hs = []
        for b in range(2):
            k = k2 * 2 + b
            o = pl.multiple_of((wid + k * NW) * GB, 8)
            pltpu.sync_copy(kj_hbm.at[pl.ds(o, GB)], idxb.at[b])
            hs.append(pltpu.async_copy(p_hbm.at[idxb.at[b]], rows.at[b], gsem))
        ws = []
        for b in range(2):
            k = k2 * 2 + b
            o = pl.multiple_of((wid + k * NW) * GB, 8)
            hs[b].wait()
            ws.append(pltpu.async_copy(rows.at[b], pk_out.at[pl.ds(o, GB)], wsem))
        for w in ws:
            w.wait()
        return 0

    lax.fori_loop(0, nb // 2, outer, 0)

    def tail(k, _):
        o = pl.multiple_of((wid + k * NW) * GB, 8)
        pltpu.sync_copy(kj_hbm.at[pl.ds(o, GB)], idxb.at[0])
        pltpu.async_copy(p_hbm.at[idxb.at[0]], rows.at[0], gsem).wait()
        pltpu.sync_copy(rows.at[0], pk_out.at[pl.ds(o, GB)])
        return 0

    lax.fori_loop(nb // 2 * 2, nb, tail, 0)


CHUNK = 6400
NCHUNK = E // CHUNK
NCITER = (NCHUNK + 1) // NC
TPW = T // NS
SLICE = CHUNK // NS
SB = 128
DUMMY = CHUNK


@functools.partial(
    pl.kernel,
    out_type=jax.ShapeDtypeStruct((E, H), jnp.float32),
    scratch_types=[
        pltpu.VMEM((TPW,), jnp.int32),
        pltpu.VMEM((TPW + SB,), jnp.int32),
        pltpu.VMEM((2, SB), jnp.int32),
        pltpu.VMEM((2, SB), jnp.int32),
        pltpu.VMEM((2, SB, H), jnp.float32),
        pltpu.VMEM((64, H), jnp.float32),
        pltpu.VMEM_SHARED((CHUNK + 8, H), jnp.float32),
        pltpu.SemaphoreType.DMA,
        pltpu.SemaphoreType.DMA,
    ],
    **_sc_mesh,
)
def _sc_scatter(tmsg_hbm, ji2_hbm, out_hbm, jibuf, pbuf, tidx, didx,
                rows, zbuf, spc, sem0, sem1):
    cid = lax.axis_index("c")
    sid = lax.axis_index("s")
    t0 = sid * TPW
    pltpu.sync_copy(ji2_hbm.at[sid], jibuf)

    zv = jnp.zeros((16,), jnp.float32)

    def zrow(r, _):
        for c in range(8):
            zbuf[r, pl.ds(c * 16, 16)] = zv
        return 0

    lax.fori_loop(0, 64, zrow, 0)
    lanes = lax.iota(jnp.int32, 16)

    def chunk_body(ci, _):
        chunk = ci * NC + cid
        start = chunk * CHUNK
        valid = chunk < NCHUNK

        @pl.when(valid)
        def _zero():
            for z in range(6):
                pltpu.sync_copy(zbuf, spc.at[pl.ds(sid * SLICE + z * 64, 64)])
            pltpu.sync_copy(zbuf.at[pl.ds(0, SLICE - 6 * 64)],
                            spc.at[pl.ds(sid * SLICE + 6 * 64, SLICE - 6 * 64)])

        plsc.subcore_barrier()

        def fbody(i, cnt_vec):
            for u in range(5):
                ii = i * 5 + u
                v = jibuf[pl.ds(ii * 16, 16)]
                msk = (v >= start) & (v < start + CHUNK)
                inc = plsc.cumsum(msk.astype(jnp.int32))
                pos = (cnt_vec - 1) + inc
                packed = ((t0 + ii * 16) + lanes) * 8192 + (v - start)
                plsc.store_scatter(pbuf, [pos], packed, mask=msk)
                cnt_vec = cnt_vec + plsc.all_reduce_population_count(msk)
            return cnt_vec

        cnt_vec = lax.fori_loop(0, TPW // 16 // 5, fbody,
                                jnp.zeros((16,), jnp.int32))
        cnt = jnp.max(cnt_vec)
        pad_base = lax.rem(t0 + TPW, jnp.int32(T))
        ones = lanes >= 0
        for k in range(SB // 16):
            pos = cnt_vec + (k * 16) + lanes
            plsc.store_scatter(
                pbuf, [pos],
                (pad_base + (k * 16) + lanes) * 8192 + DUMMY, mask=ones)

        nbp = jnp.maximum((cnt + SB - 1) // SB, 1)
        sems = (sem0, sem1)

        def prep(b, j):
            o = pl.multiple_of(j * SB, 8)
            for k in range(SB // 16):
                w = pbuf[pl.ds(o + k * 16, 16)]
                tidx[b, pl.ds(k * 16, 16)] = jnp.right_shift(w, 13)
                didx[b, pl.ds(k * 16, 16)] = jnp.bitwise_and(w, 8191)

        def fire(b, j):
            prep(b, j)
            pltpu.async_copy(tmsg_hbm.at[tidx.at[b]], rows.at[b], sems[b])

        def drain_accum(b):
            pltpu.make_async_copy(tmsg_hbm.at[pl.ds(0, SB)], rows.at[b],
                                  sems[b]).wait()
            pltpu.sync_copy(rows.at[b], spc.at[didx.at[b]], add=True)

        fire(0, jnp.int32(0))

        def bpair(q, _):
            j1 = q * 2 + 1

            @pl.when(j1 < nbp)
            def _fire1():
                fire(1, j1)

            drain_accum(0)

            @pl.when(j1 + 1 < nbp)
            def _fire0():
                fire(0, j1 + 1)

            @pl.when(j1 < nbp)
            def _drain1():
                drain_accum(1)

            return 0

        lax.fori_loop(0, (nbp + 1) // 2, bpair, 0)
        plsc.subcore_barrier()

        @pl.when(valid)
        def _writeout():
            pltpu.sync_copy(
                spc.at[pl.ds(sid * SLICE, SLICE)],
                out_hbm.at[pl.ds(pl.multiple_of(start + sid * SLICE, 8),
                                 SLICE)])

        return 0

    lax.fori_loop(0, NCITER, chunk_body, 0)


TM = 6400


def _silu(x):
    return x * (1.0 / (1.0 + jnp.exp(-x)))


def _dotT(at, w):
    return lax.dot_general(at, w, (((0,), (0,)), ((), ())),
                           preferred_element_type=jnp.float32)


def _tc_pre_body(m, rbf_t, w_kj, b_kj, w_rbf, p):
    x = _silu(jnp.dot(m[:].astype(jnp.bfloat16), w_kj[:],
                      preferred_element_type=jnp.float32) + b_kj[:])
    gate = _dotT(rbf_t[:], w_rbf[:])
    p[:] = x * gate


def _tc_pre(m, rbf_t, w_kj, b_kj, w_rbf):
    full = lambda *shape: pl.BlockSpec(shape, lambda i: tuple(0 for _ in shape))
    return pl.pallas_call(
        _tc_pre_body,
        grid=(E // TM,),
        in_specs=[
            pl.BlockSpec((TM, H), lambda i: (i, 0)),
            pl.BlockSpec((6, TM), lambda i: (0, i)),
            full(H, H), full(1, H), full(6, H),
        ],
        out_specs=pl.BlockSpec((TM, H), lambda i: (i, 0)),
        out_shape=jax.ShapeDtypeStruct((E, H), jnp.float32),
    )(m, rbf_t, w_kj, b_kj, w_rbf)


def _tc_triplet_body(pk, sbf_t, w_sbf, wb, tmsg):
    sp = _dotT(sbf_t[:], w_sbf[:])
    xk = pk[:].astype(jnp.bfloat16)
    acc = jnp.zeros((TM, H), jnp.float32)
    for b in range(NB):
        acc = acc + jnp.dot(xk, wb[b], preferred_element_type=jnp.float32) \
            * sp[:, b:b + 1]
    tmsg[:] = acc


def _tc_triplet(pk, sbf_t, w_sbf, wb):
    full = lambda *shape: pl.BlockSpec(shape, lambda i: tuple(0 for _ in shape))
    return pl.pallas_call(
        _tc_triplet_body,
        grid=(T // TM,),
        in_specs=[
            pl.BlockSpec((TM, H), lambda i: (i, 0)),
            pl.BlockSpec((42, TM), lambda i: (0, i)),
            full(42, NB), full(NB, H, H),
        ],
        out_specs=pl.BlockSpec((TM, H), lambda i: (i, 0)),
        out_shape=jax.ShapeDtypeStruct((T, H), jnp.float32),
    )(pk, sbf_t, w_sbf, wb)


def _bdot(a, w):
    return jnp.dot(a.astype(jnp.bfloat16), w,
                   preferred_element_type=jnp.float32)


def _tc_post_body(m, aggr, w_ji, b_ji, r1w1, r1b1, r1w2, r1b2,
                  r2w1, r2b1, r2w2, r2b2, w_out, b_out, out):
    mv = m[:]
    x = _silu(_bdot(mv, w_ji[:]) + b_ji[:]) + aggr[:]
    h = _silu(x)
    h = _silu(_bdot(h, r1w1[:]) + r1b1[:])
    h = _bdot(h, r1w2[:]) + r1b2[:]
    x = x + h
    h = _silu(x)
    h = _silu(_bdot(h, r2w1[:]) + r2b1[:])
    h = _bdot(h, r2w2[:]) + r2b2[:]
    x = x + h
    out[:] = mv + _silu(_bdot(x, w_out[:]) + b_out[:])


def _tc_post(m, aggr, w_ji, b_ji, r1w1, r1b1, r1w2, r1b2,
             r2w1, r2b1, r2w2, r2b2, w_out, b_out):
    mat = pl.BlockSpec((H, H), lambda i: (0, 0))
    vec = pl.BlockSpec((1, H), lambda i: (0, 0))
    return pl.pallas_call(
        _tc_post_body,
        grid=(E // TM,),
        in_specs=[
            pl.BlockSpec((TM, H), lambda i: (i, 0)),
            pl.BlockSpec((TM, H), lambda i: (i, 0)),
            mat, vec, mat, vec, mat, vec, mat, vec, mat, vec, mat, vec,
        ],
        out_specs=pl.BlockSpec((TM, H), lambda i: (i, 0)),
        out_shape=jax.ShapeDtypeStruct((E, H), jnp.float32),
    )(m, aggr, w_ji, b_ji, r1w1, r1b1, r1w2, r1b2,
      r2w1, r2b1, r2w2, r2b2, w_out, b_out)


def kernel(m, rbf, sbf, edge_index, triplet_index, W_rbf, W_sbf, W_kj, b_kj,
           W_ji, b_ji, W_bil, r1_w1, r1_b1, r1_w2, r1_b2, r2_w1, r2_b1,
           r2_w2, r2_b2, W_out, b_out):
    bf16 = jnp.bfloat16
    kj = triplet_index[0].astype(jnp.int32)
    ji = triplet_index[1].astype(jnp.int32)
    ji2 = ji.reshape(NS, TPW)
    wb = jnp.transpose(W_bil, (1, 0, 2)).astype(bf16)

    p = _tc_pre(m, rbf.T, W_kj.astype(bf16), b_kj.reshape(1, H), W_rbf)
    pk = _sc_gather(p, kj)
    tmsg = _tc_triplet(pk, sbf.T, W_sbf, wb)
    aggr = _sc_scatter(tmsg, ji2)
    return _tc_post(m, aggr, W_ji.astype(bf16), b_ji.reshape(1, H),
                    r1_w1.astype(bf16), r1_b1.reshape(1, H),
                    r1_w2.astype(bf16), r1_b2.reshape(1, H),
                    r2_w1.astype(bf16), r2_b1.reshape(1, H),
                    r2_w2.astype(bf16), r2_b2.reshape(1, H),
                    W_out.astype(bf16), b_out.reshape(1, H))


# speedup vs baseline: 1.4434x; 1.0769x over previous
import functools

import jax
import jax.numpy as jnp
from jax import lax
from jax.experimental import pallas as pl
from jax.experimental.pallas import tpu as pltpu
from jax.experimental.pallas import tpu_sc as plsc

E = 160000
T = 160000
H = 128
NB = 8

NC = 2
NS = 16
NW = NC * NS

GB = 128
GNB = T // GB

_sc_mesh = dict(
    mesh=plsc.VectorSubcoreMesh(core_axis_name="c", subcore_axis_name="s"),
    compiler_params=pltpu.CompilerParams(needs_layout_passes=False),
)


@functools.partial(
    pl.kernel,
    out_type=jax.ShapeDtypeStruct((T, H), jnp.float32),
    scratch_types=[
        pltpu.VMEM((2, GB), jnp.int32),
        pltpu.VMEM((2, GB, H), jnp.float32),
        pltpu.SemaphoreType.DMA,
        pltpu.SemaphoreType.DMA,
    ],
    **_sc_mesh,
)
def _sc_gather(p_hbm, kj_hbm, pk_out, idxb, rows, gsem, wsem):
    wid = lax.axis_index("s") * NC + lax.axis_index("c")
    nb = jnp.where(wid < GNB % NW, GNB // NW + 1, GNB // NW)

    def outer(k2, _):
        hs = []
        for b in range(2):
            k = k2 * 2 + b
            o = pl.multiple_of((wid + k * NW) * GB, 8)
            pltpu.sync_copy(kj_hbm.at[pl.ds(o, GB)], idxb.at[b])
            hs.append(pltpu.async_copy(p_hbm.at[idxb.at[b]], rows.at[b], gsem))
        ws = []
        for b in range(2):
            k = k2 * 2 + b
            o = pl.multiple_of((wid + k * NW) * GB, 8)
            hs[b].wait()
            ws.append(pltpu.async_copy(rows.at[b], pk_out.at[pl.ds(o, GB)], wsem))
        for w in ws:
            w.wait()
        return 0

    lax.fori_loop(0, nb // 2, outer, 0)

    def tail(k, _):
        o = pl.multiple_of((wid + k * NW) * GB, 8)
        pltpu.sync_copy(kj_hbm.at[pl.ds(o, GB)], idxb.at[0])
        pltpu.async_copy(p_hbm.at[idxb.at[0]], rows.at[0], gsem).wait()
        pltpu.sync_copy(rows.at[0], pk_out.at[pl.ds(o, GB)])
        return 0

    lax.fori_loop(nb // 2 * 2, nb, tail, 0)


CHUNK = 6400
NCHUNK = E // CHUNK
NCITER = (NCHUNK + 1) // NC
TPW = T // NS
SLICE = CHUNK // NS
SB = 128
DUMMY = CHUNK


@functools.partial(
    pl.kernel,
    out_type=jax.ShapeDtypeStruct((E, H), jnp.float32),
    scratch_types=[
        pltpu.VMEM((TPW,), jnp.int32),
        pltpu.VMEM((TPW + SB,), jnp.int32),
        pltpu.VMEM((2, SB), jnp.int32),
        pltpu.VMEM((2, SB), jnp.int32),
        pltpu.VMEM((2, SB, H), jnp.float32),
        pltpu.VMEM((64, H), jnp.float32),
        pltpu.VMEM_SHARED((CHUNK + 8, H), jnp.float32),
        pltpu.SemaphoreType.DMA,
        pltpu.SemaphoreType.DMA,
        pltpu.SemaphoreType.DMA,
    ],
    **_sc_mesh,
)
def _sc_scatter(tmsg_hbm, ji2_hbm, out_hbm, jibuf, pbuf, tidx, didx,
                rows, zbuf, spc, sem0, sem1, wsem):
    cid = lax.axis_index("c")
    sid = lax.axis_index("s")
    t0 = sid * TPW
    pltpu.sync_copy(ji2_hbm.at[sid], jibuf)

    zv = jnp.zeros((16,), jnp.float32)

    def zrow(r, _):
        for c in range(8):
            zbuf[r, pl.ds(c * 16, 16)] = zv
        return 0

    lax.fori_loop(0, 64, zrow, 0)
    lanes = lax.iota(jnp.int32, 16)

    def chunk_body(ci, _):
        chunk = ci * NC + cid
        start = chunk * CHUNK
        valid = chunk < NCHUNK

        def fbody(i, cnt_vec):
            for u in range(5):
                ii = i * 5 + u
                v = jibuf[pl.ds(ii * 16, 16)]
                msk = (v >= start) & (v < start + CHUNK)
                inc = plsc.cumsum(msk.astype(jnp.int32))
                pos = (cnt_vec - 1) + inc
                packed = ((t0 + ii * 16) + lanes) * 8192 + (v - start)
                plsc.store_scatter(pbuf, [pos], packed, mask=msk)
                cnt_vec = cnt_vec + plsc.all_reduce_population_count(msk)
            return cnt_vec

        cnt_vec = lax.fori_loop(0, TPW // 16 // 5, fbody,
                                jnp.zeros((16,), jnp.int32))
        cnt = jnp.max(cnt_vec)
        pad_base = lax.rem(t0 + TPW, jnp.int32(T))
        ones = lanes >= 0
        for k in range(SB // 16):
            pos = cnt_vec + (k * 16) + lanes
            plsc.store_scatter(
                pbuf, [pos],
                (pad_base + (k * 16) + lanes) * 8192 + DUMMY, mask=ones)

        @pl.when(valid)
        def _zero():
            @pl.when(ci > 0)
            def _drain_writeout():
                pltpu.make_async_copy(
                    spc.at[pl.ds(sid * SLICE, SLICE)],
                    out_hbm.at[pl.ds(pl.multiple_of(sid * SLICE, 8), SLICE)],
                    wsem).wait()

            hs = [pltpu.async_copy(
                zbuf, spc.at[pl.ds(sid * SLICE + z * 64, 64)], sem0)
                for z in range(6)]
            hs.append(pltpu.async_copy(
                zbuf.at[pl.ds(0, SLICE - 6 * 64)],
                spc.at[pl.ds(sid * SLICE + 6 * 64, SLICE - 6 * 64)], sem0))
            for h in hs:
                h.wait()

        plsc.subcore_barrier()

        nbp = jnp.maximum((cnt + SB - 1) // SB, 1)
        sems = (sem0, sem1)

        def prep(b, j):
            o = pl.multiple_of(j * SB, 8)
            for k in range(SB // 16):
                w = pbuf[pl.ds(o + k * 16, 16)]
                tidx[b, pl.ds(k * 16, 16)] = jnp.right_shift(w, 13)
                didx[b, pl.ds(k * 16, 16)] = jnp.bitwise_and(w, 8191)

        def fire(b, j):
            prep(b, j)
            pltpu.async_copy(tmsg_hbm.at[tidx.at[b]], rows.at[b], sems[b])

        def drain_accum(b):
            pltpu.make_async_copy(tmsg_hbm.at[pl.ds(0, SB)], rows.at[b],
                                  sems[b]).wait()
            pltpu.sync_copy(rows.at[b], spc.at[didx.at[b]], add=True)

        fire(0, jnp.int32(0))

        def bpair(q, _):
            j1 = q * 2 + 1

            @pl.when(j1 < nbp)
            def _fire1():
                fire(1, j1)

            drain_accum(0)

            @pl.when(j1 + 1 < nbp)
            def _fire0():
                fire(0, j1 + 1)

            @pl.when(j1 < nbp)
            def _drain1():
                drain_accum(1)

            return 0

        lax.fori_loop(0, (nbp + 1) // 2, bpair, 0)
        plsc.subcore_barrier()

        @pl.when(valid)
        def _writeout():
            pltpu.async_copy(
                spc.at[pl.ds(sid * SLICE, SLICE)],
                out_hbm.at[pl.ds(pl.multiple_of(start + sid * SLICE, 8),
                                 SLICE)], wsem)

        return 0

    lax.fori_loop(0, NCITER, chunk_body, 0)
    pltpu.make_async_copy(
        spc.at[pl.ds(sid * SLICE, SLICE)],
        out_hbm.at[pl.ds(pl.multiple_of(sid * SLICE, 8), SLICE)],
        wsem).wait()


TM = 6400


def _silu(x):
    return x * (1.0 / (1.0 + jnp.exp(-x)))


def _dotT(at, w):
    return lax.dot_general(at, w, (((0,), (0,)), ((), ())),
                           preferred_element_type=jnp.float32)


def _tc_pre_body(m, rbf_t, w_kj, b_kj, w_rbf, p):
    x = _silu(jnp.dot(m[:].astype(jnp.bfloat16), w_kj[:],
                      preferred_element_type=jnp.float32) + b_kj[:])
    gate = _dotT(rbf_t[:], w_rbf[:])
    p[:] = x * gate


def _tc_pre(m, rbf_t, w_kj, b_kj, w_rbf):
    full = lambda *shape: pl.BlockSpec(shape, lambda i: tuple(0 for _ in shape))
    return pl.pallas_call(
        _tc_pre_body,
        grid=(E // TM,),
        in_specs=[
            pl.BlockSpec((TM, H), lambda i: (i, 0)),
            pl.BlockSpec((6, TM), lambda i: (0, i)),
            full(H, H), full(1, H), full(6, H),
        ],
        out_specs=pl.BlockSpec((TM, H), lambda i: (i, 0)),
        out_shape=jax.ShapeDtypeStruct((E, H), jnp.float32),
    )(m, rbf_t, w_kj, b_kj, w_rbf)


def _tc_triplet_body(pk, sbf_t, w_sbf, wb, tmsg):
    sp = _dotT(sbf_t[:], w_sbf[:])
    xk = pk[:].astype(jnp.bfloat16)
    acc = jnp.zeros((TM, H), jnp.float32)
    for b in range(NB):
        acc = acc + jnp.dot(xk, wb[b], preferred_element_type=jnp.float32) \
            * sp[:, b:b + 1]
    tmsg[:] = acc


def _tc_triplet(pk, sbf_t, w_sbf, wb):
    full = lambda *shape: pl.BlockSpec(shape, lambda i: tuple(0 for _ in shape))
    return pl.pallas_call(
        _tc_triplet_body,
        grid=(T // TM,),
        in_specs=[
            pl.BlockSpec((TM, H), lambda i: (i, 0)),
            pl.BlockSpec((42, TM), lambda i: (0, i)),
            full(42, NB), full(NB, H, H),
        ],
        out_specs=pl.BlockSpec((TM, H), lambda i: (i, 0)),
        out_shape=jax.ShapeDtypeStruct((T, H), jnp.float32),
    )(pk, sbf_t, w_sbf, wb)


def _bdot(a, w):
    return jnp.dot(a.astype(jnp.bfloat16), w,
                   preferred_element_type=jnp.float32)


def _tc_post_body(m, aggr, w_ji, b_ji, r1w1, r1b1, r1w2, r1b2,
                  r2w1, r2b1, r2w2, r2b2, w_out, b_out, out):
    mv = m[:]
    x = _silu(_bdot(mv, w_ji[:]) + b_ji[:]) + aggr[:]
    h = _silu(x)
    h = _silu(_bdot(h, r1w1[:]) + r1b1[:])
    h = _bdot(h, r1w2[:]) + r1b2[:]
    x = x + h
    h = _silu(x)
    h = _silu(_bdot(h, r2w1[:]) + r2b1[:])
    h = _bdot(h, r2w2[:]) + r2b2[:]
    x = x + h
    out[:] = mv + _silu(_bdot(x, w_out[:]) + b_out[:])


def _tc_post(m, aggr, w_ji, b_ji, r1w1, r1b1, r1w2, r1b2,
             r2w1, r2b1, r2w2, r2b2, w_out, b_out):
    mat = pl.BlockSpec((H, H), lambda i: (0, 0))
    vec = pl.BlockSpec((1, H), lambda i: (0, 0))
    return pl.pallas_call(
        _tc_post_body,
        grid=(E // TM,),
        in_specs=[
            pl.BlockSpec((TM, H), lambda i: (i, 0)),
            pl.BlockSpec((TM, H), lambda i: (i, 0)),
            mat, vec, mat, vec, mat, vec, mat, vec, mat, vec, mat, vec,
        ],
        out_specs=pl.BlockSpec((TM, H), lambda i: (i, 0)),
        out_shape=jax.ShapeDtypeStruct((E, H), jnp.float32),
    )(m, aggr, w_ji, b_ji, r1w1, r1b1, r1w2, r1b2,
      r2w1, r2b1, r2w2, r2b2, w_out, b_out)


def kernel(m, rbf, sbf, edge_index, triplet_index, W_rbf, W_sbf, W_kj, b_kj,
           W_ji, b_ji, W_bil, r1_w1, r1_b1, r1_w2, r1_b2, r2_w1, r2_b1,
           r2_w2, r2_b2, W_out, b_out):
    bf16 = jnp.bfloat16
    kj = triplet_index[0].astype(jnp.int32)
    ji = triplet_index[1].astype(jnp.int32)
    ji2 = ji.reshape(NS, TPW)
    wb = jnp.transpose(W_bil, (1, 0, 2)).astype(bf16)

    p = _tc_pre(m, rbf.T, W_kj.astype(bf16), b_kj.reshape(1, H), W_rbf)
    pk = _sc_gather(p, kj)
    tmsg = _tc_triplet(pk, sbf.T, W_sbf, wb)
    aggr = _sc_scatter(tmsg, ji2)
    return _tc_post(m, aggr, W_ji.astype(bf16), b_ji.reshape(1, H),
                    r1_w1.astype(bf16), r1_b1.reshape(1, H),
                    r1_w2.astype(bf16), r1_b2.reshape(1, H),
                    r2_w1.astype(bf16), r2_b1.reshape(1, H),
                    r2_w2.astype(bf16), r2_b2.reshape(1, H),
                    W_out.astype(bf16), b_out.reshape(1, H))
